# baseline (speedup 1.0000x reference)
import functools
import jax
import jax.numpy as jnp
from jax.experimental import pallas as pl
from jax.experimental.pallas import tpu as pltpu

_C = 256
_GROUPS = 32
_GSIZE = _C // _GROUPS
_EPS = 1e-5


def _group_mat():
    r = jax.lax.broadcasted_iota(jnp.int32, (_C, _C), 0) // _GSIZE
    c = jax.lax.broadcasted_iota(jnp.int32, (_C, _C), 1) // _GSIZE
    return (r == c).astype(jnp.float32)


def _tower_kernel(*refs, H, W, n_layers, head_co, bbox):
    if bbox:
        (feat_ref, tw_ref, lp_ref, hw_ref, hb_ref, sc_ref, out_ref,
         pad_ref) = refs
    else:
        feat_ref, tw_ref, lp_ref, hw_ref, hb_ref, out_ref, pad_ref = refs
    N = H * W
    P = _group_mat()

    pad_ref[...] = jnp.zeros_like(pad_ref)
    pad_ref[1:H + 1, 1:W + 1, :] = feat_ref[0]

    def im2col():
        return jnp.concatenate(
            [pad_ref[k // 3:k // 3 + H, k % 3:k % 3 + W, :].reshape(N, _C)
             for k in range(9)], axis=1)

    for layer in range(n_layers):
        acc = jnp.dot(im2col(), tw_ref[layer],
                      preferred_element_type=jnp.float32)
        lp = lp_ref[layer]
        b, gamma, beta = lp[0:1], lp[1:2], lp[2:3]
        gsb, gsb2 = lp[3:4], lp[4:5]
        s = jnp.sum(acc, axis=0, keepdims=True)
        q = jnp.sum(acc * acc, axis=0, keepdims=True)
        stats = jnp.concatenate([s, q, b * s], axis=0)
        gs = jnp.dot(stats, P, preferred_element_type=jnp.float32)
        inv_n = 1.0 / (_GSIZE * N)
        mu = (gs[0:1] + N * gsb) * inv_n
        ey2 = (gs[1:2] + 2.0 * gs[2:3] + N * gsb2) * inv_n
        rstd = jax.lax.rsqrt(ey2 - mu * mu + _EPS)
        sc = rstd * gamma
        sh = (b - mu) * sc + beta
        x = jnp.maximum(acc * sc + sh, 0.0).astype(jnp.bfloat16)
        pad_ref[1:H + 1, 1:W + 1, :] = x.reshape(H, W, _C)

    y = jnp.dot(im2col(), hw_ref[...],
                preferred_element_type=jnp.float32) + hb_ref[0:1]
    if bbox:
        y = jnp.exp(y * sc_ref[...])
    out_ref[0] = y.reshape(H, W, head_co)


def _run_tower(feat, tower_w, lp, head_w, head_b, scale, head_co, bbox):
    B, H, W, _ = feat.shape
    kern = functools.partial(_tower_kernel, H=H, W=W,
                             n_layers=tower_w.shape[0],
                             head_co=head_co, bbox=bbox)
    in_specs = [
        pl.BlockSpec((1, H, W, _C), lambda b: (b, 0, 0, 0)),
        pl.BlockSpec(tower_w.shape, lambda b: (0, 0, 0)),
        pl.BlockSpec(lp.shape, lambda b: (0, 0, 0)),
        pl.BlockSpec(head_w.shape, lambda b: (0, 0)),
        pl.BlockSpec(head_b.shape, lambda b: (0, 0)),
    ]
    args = [feat, tower_w, lp, head_w, head_b]
    if bbox:
        in_specs.append(pl.BlockSpec((1, 1), lambda b: (0, 0)))
        args.append(scale)
    return pl.pallas_call(
        kern,
        grid=(B,),
        in_specs=in_specs,
        out_specs=pl.BlockSpec((1, H, W, head_co), lambda b: (b, 0, 0, 0)),
        out_shape=jax.ShapeDtypeStruct((B, H, W, head_co), jnp.float32),
        scratch_shapes=[pltpu.VMEM((H + 2, W + 2, _C), jnp.bfloat16)],
    )(*args)


def _gs_vec(v):
    return jnp.repeat(v.reshape(_GROUPS, _GSIZE).sum(axis=1), _GSIZE)


def _prep_tower(layers):
    ws, lps = [], []
    for l in layers:
        ws.append(jnp.transpose(l['w'], (2, 3, 1, 0)).reshape(9 * _C, _C))
        b, g, beta = l['b'], l['g'], l['beta']
        lps.append(jnp.stack([b, g, beta, _gs_vec(b), _gs_vec(b * b),
                              jnp.zeros_like(b), jnp.zeros_like(b),
                              jnp.zeros_like(b)]))
    return (jnp.stack(ws).astype(jnp.bfloat16),
            jnp.stack(lps).astype(jnp.float32))


def _prep_head(w):
    co = w.shape[0]
    return jnp.transpose(w, (2, 3, 1, 0)).reshape(9 * _C, co).astype(
        jnp.bfloat16)


def kernel(features, params):
    cls_tw, cls_lp = _prep_tower(params['cls_tower'])
    box_tw, box_lp = _prep_tower(params['bbox_tower'])
    cls_head_w = _prep_head(jnp.concatenate(
        [params['cls_logits']['w'], params['centerness']['w']], axis=0))
    cls_head_b = jnp.concatenate(
        [params['cls_logits']['b'], params['centerness']['b']])[None, :]
    box_head_w = _prep_head(params['bbox_pred']['w'])
    box_head_b = params['bbox_pred']['b'][None, :]

    logits, bbox, ctr = [], [], []
    for l, f in enumerate(features):
        fx = jnp.transpose(f, (0, 2, 3, 1)).astype(jnp.bfloat16)
        yc = _run_tower(fx, cls_tw, cls_lp, cls_head_w, cls_head_b,
                        None, 81, False)
        sc = params['scales'][l].reshape(1, 1)
        yb = _run_tower(fx, box_tw, box_lp, box_head_w, box_head_b,
                        sc, 4, True)
        logits.append(jnp.transpose(yc[..., :80], (0, 3, 1, 2)))
        ctr.append(jnp.transpose(yc[..., 80:81], (0, 3, 1, 2)))
        bbox.append(jnp.transpose(yb, (0, 3, 1, 2)))
    return tuple(logits), tuple(bbox), tuple(ctr)


# speedup vs baseline: 1.3500x; 1.3500x over previous
import functools
import jax
import jax.numpy as jnp
from jax.experimental import pallas as pl
from jax.experimental.pallas import tpu as pltpu

_C = 256
_GROUPS = 32
_GSIZE = _C // _GROUPS
_EPS = 1e-5


def _group_mat():
    r = jax.lax.broadcasted_iota(jnp.int32, (_C, _C), 0) // _GSIZE
    c = jax.lax.broadcasted_iota(jnp.int32, (_C, _C), 1) // _GSIZE
    return (r == c).astype(jnp.float32)


def _chunks(H, W):
    ch = max(1, min(H, 512 // W))
    return [(h0, min(ch, H - h0)) for h0 in range(0, H, ch)]


def _im2col(pad_ref, h0, ch, W):
    return jnp.concatenate(
        [pad_ref[h0 + k // 3:h0 + k // 3 + ch,
                 k % 3:k % 3 + W, :].reshape(ch * W, _C)
         for k in range(9)], axis=1)


def _tower(feat_ref, tw_ref, lp_ref, hw_ref, hb_ref, out_ref, pad_ref,
           acc_ref, sc_ref, H, W, head_co, bbox):
    N = H * W
    P = _group_mat()
    chunks = _chunks(H, W)

    pad_ref[...] = jnp.zeros_like(pad_ref)
    pad_ref[1:H + 1, 1:W + 1, :] = feat_ref[0]

    for layer in range(tw_ref.shape[0]):
        for h0, ch in chunks:
            acc_ref[h0 * W:(h0 + ch) * W, :] = jnp.dot(
                _im2col(pad_ref, h0, ch, W), tw_ref[layer],
                preferred_element_type=jnp.float32)
        acc = acc_ref[0:N, :]
        lp = lp_ref[layer]
        b, gamma, beta = lp[0:1], lp[1:2], lp[2:3]
        gsb, gsb2 = lp[3:4], lp[4:5]
        s = jnp.sum(acc, axis=0, keepdims=True)
        q = jnp.sum(acc * acc, axis=0, keepdims=True)
        stats = jnp.concatenate([s, q, b * s], axis=0)
        gs = jnp.dot(stats, P, preferred_element_type=jnp.float32)
        inv_n = 1.0 / (_GSIZE * N)
        mu = (gs[0:1] + N * gsb) * inv_n
        ey2 = (gs[1:2] + 2.0 * gs[2:3] + N * gsb2) * inv_n
        rstd = jax.lax.rsqrt(ey2 - mu * mu + _EPS)
        sc = rstd * gamma
        sh = (b - mu) * sc + beta
        x = jnp.maximum(acc * sc + sh, 0.0).astype(jnp.bfloat16)
        pad_ref[1:H + 1, 1:W + 1, :] = x.reshape(H, W, _C)

    for h0, ch in chunks:
        y = jnp.dot(_im2col(pad_ref, h0, ch, W), hw_ref[...],
                    preferred_element_type=jnp.float32) + hb_ref[0:1]
        if bbox:
            y = jnp.exp(y * sc_ref[...])
        out_ref[0, h0:h0 + ch] = y.reshape(ch, W, head_co)


def _level_kernel(feat_ref, ctw_ref, clp_ref, chw_ref, chb_ref,
                  btw_ref, blp_ref, bhw_ref, bhb_ref, sc_ref,
                  cls_out, box_out, pad_ref, acc_ref, *, H, W):
    _tower(feat_ref, ctw_ref, clp_ref, chw_ref, chb_ref, cls_out,
           pad_ref, acc_ref, None, H, W, 81, False)
    _tower(feat_ref, btw_ref, blp_ref, bhw_ref, bhb_ref, box_out,
           pad_ref, acc_ref, sc_ref, H, W, 4, True)


def _run_level(feat, cls_p, box_p, scale):
    B, H, W, _ = feat.shape
    cls_tw, cls_lp, cls_hw, cls_hb = cls_p
    box_tw, box_lp, box_hw, box_hb = box_p
    kern = functools.partial(_level_kernel, H=H, W=W)
    full = lambda a: pl.BlockSpec(a.shape, lambda b: (0,) * a.ndim)
    args = [cls_tw, cls_lp, cls_hw, cls_hb,
            box_tw, box_lp, box_hw, box_hb, scale]
    in_specs = ([pl.BlockSpec((1, H, W, _C), lambda b: (b, 0, 0, 0))]
                + [full(a) for a in args])
    return pl.pallas_call(
        kern,
        grid=(B,),
        in_specs=in_specs,
        out_specs=[
            pl.BlockSpec((1, H, W, 81), lambda b: (b, 0, 0, 0)),
            pl.BlockSpec((1, H, W, 4), lambda b: (b, 0, 0, 0)),
        ],
        out_shape=[
            jax.ShapeDtypeStruct((B, H, W, 81), jnp.float32),
            jax.ShapeDtypeStruct((B, H, W, 4), jnp.float32),
        ],
        scratch_shapes=[
            pltpu.VMEM((H + 2, W + 2, _C), jnp.bfloat16),
            pltpu.VMEM((H * W, _C), jnp.float32),
        ],
    )(feat, *args)


def _gs_vec(v):
    return jnp.repeat(v.reshape(_GROUPS, _GSIZE).sum(axis=1), _GSIZE)


def _prep_tower(layers):
    ws, lps = [], []
    for l in layers:
        ws.append(jnp.transpose(l['w'], (2, 3, 1, 0)).reshape(9 * _C, _C))
        b, g, beta = l['b'], l['g'], l['beta']
        lps.append(jnp.stack([b, g, beta, _gs_vec(b), _gs_vec(b * b),
                              jnp.zeros_like(b), jnp.zeros_like(b),
                              jnp.zeros_like(b)]))
    return (jnp.stack(ws).astype(jnp.bfloat16),
            jnp.stack(lps).astype(jnp.float32))


def _prep_head(w):
    co = w.shape[0]
    return jnp.transpose(w, (2, 3, 1, 0)).reshape(9 * _C, co).astype(
        jnp.bfloat16)


def kernel(features, params):
    cls_tw, cls_lp = _prep_tower(params['cls_tower'])
    box_tw, box_lp = _prep_tower(params['bbox_tower'])
    cls_hw = _prep_head(jnp.concatenate(
        [params['cls_logits']['w'], params['centerness']['w']], axis=0))
    cls_hb = jnp.concatenate(
        [params['cls_logits']['b'], params['centerness']['b']])[None, :]
    box_hw = _prep_head(params['bbox_pred']['w'])
    box_hb = params['bbox_pred']['b'][None, :]
    cls_p = (cls_tw, cls_lp, cls_hw, cls_hb)
    box_p = (box_tw, box_lp, box_hw, box_hb)

    logits, bbox, ctr = [], [], []
    for l, f in enumerate(features):
        fx = jnp.transpose(f, (0, 2, 3, 1)).astype(jnp.bfloat16)
        sc = params['scales'][l].reshape(1, 1)
        yc, yb = _run_level(fx, cls_p, box_p, sc)
        logits.append(jnp.transpose(yc[..., :80], (0, 3, 1, 2)))
        ctr.append(jnp.transpose(yc[..., 80:81], (0, 3, 1, 2)))
        bbox.append(jnp.transpose(yb, (0, 3, 1, 2)))
    return tuple(logits), tuple(bbox), tuple(ctr)


# speedup vs baseline: 1.4128x; 1.0465x over previous
import functools
import jax
import jax.numpy as jnp
from jax.experimental import pallas as pl
from jax.experimental.pallas import tpu as pltpu

_C = 256
_GROUPS = 32
_GSIZE = _C // _GROUPS
_EPS = 1e-5


def _group_mat():
    r = jax.lax.broadcasted_iota(jnp.int32, (_C, _C), 0) // _GSIZE
    c = jax.lax.broadcasted_iota(jnp.int32, (_C, _C), 1) // _GSIZE
    return (r == c).astype(jnp.float32)


def _chunks(H, W):
    ch = max(1, min(H, 256 // W))
    return [(h0, min(ch, H - h0)) for h0 in range(0, H, ch)]


def _im2col(pad_ref, h0, ch, W):
    return jnp.concatenate(
        [pad_ref[h0 + k // 3:h0 + k // 3 + ch,
                 k % 3:k % 3 + W, :].reshape(ch * W, _C)
         for k in range(9)], axis=1)


def _tower(feat_ref, tw_ref, lp_ref, hw_ref, hb_ref, out_ref, pad_ref,
           acc_ref, sc_ref, H, W, head_co, bbox):
    N = H * W
    P = _group_mat()
    chunks = _chunks(H, W)

    pad_ref[...] = jnp.zeros_like(pad_ref)
    pad_ref[1:H + 1, 1:W + 1, :] = feat_ref[0]

    for layer in range(tw_ref.shape[0]):
        s = q = None
        for h0, ch in chunks:
            a = jnp.dot(_im2col(pad_ref, h0, ch, W), tw_ref[layer],
                        preferred_element_type=jnp.float32)
            acc_ref[h0 * W:(h0 + ch) * W, :] = a
            cs = jnp.sum(a, axis=0, keepdims=True)
            cq = jnp.sum(a * a, axis=0, keepdims=True)
            s = cs if s is None else s + cs
            q = cq if q is None else q + cq
        lp = lp_ref[layer]
        b, gamma, beta = lp[0:1], lp[1:2], lp[2:3]
        gsb, gsb2 = lp[3:4], lp[4:5]
        stats = jnp.concatenate([s, q, b * s], axis=0)
        gs = jnp.dot(stats, P, preferred_element_type=jnp.float32)
        inv_n = 1.0 / (_GSIZE * N)
        mu = (gs[0:1] + N * gsb) * inv_n
        ey2 = (gs[1:2] + 2.0 * gs[2:3] + N * gsb2) * inv_n
        rstd = jax.lax.rsqrt(ey2 - mu * mu + _EPS)
        sc = rstd * gamma
        sh = (b - mu) * sc + beta
        for h0, ch in chunks:
            a = acc_ref[h0 * W:(h0 + ch) * W, :]
            x = jnp.maximum(a * sc + sh, 0.0).astype(jnp.bfloat16)
            pad_ref[h0 + 1:h0 + ch + 1, 1:W + 1, :] = x.reshape(ch, W, _C)

    for h0, ch in chunks:
        y = jnp.dot(_im2col(pad_ref, h0, ch, W), hw_ref[...],
                    preferred_element_type=jnp.float32) + hb_ref[0:1]
        if bbox:
            y = jnp.exp(y * sc_ref[...])
        out_ref[0, h0:h0 + ch] = y.reshape(ch, W, head_co)


def _level_kernel(feat_ref, ctw_ref, clp_ref, chw_ref, chb_ref,
                  btw_ref, blp_ref, bhw_ref, bhb_ref, sc_ref,
                  cls_out, box_out, pad_ref, acc_ref, *, H, W):
    _tower(feat_ref, ctw_ref, clp_ref, chw_ref, chb_ref, cls_out,
           pad_ref, acc_ref, None, H, W, 81, False)
    _tower(feat_ref, btw_ref, blp_ref, bhw_ref, bhb_ref, box_out,
           pad_ref, acc_ref, sc_ref, H, W, 4, True)


def _run_level(feat, cls_p, box_p, scale):
    B, H, W, _ = feat.shape
    cls_tw, cls_lp, cls_hw, cls_hb = cls_p
    box_tw, box_lp, box_hw, box_hb = box_p
    kern = functools.partial(_level_kernel, H=H, W=W)
    full = lambda a: pl.BlockSpec(a.shape, lambda b: (0,) * a.ndim)
    args = [cls_tw, cls_lp, cls_hw, cls_hb,
            box_tw, box_lp, box_hw, box_hb, scale]
    in_specs = ([pl.BlockSpec((1, H, W, _C), lambda b: (b, 0, 0, 0))]
                + [full(a) for a in args])
    return pl.pallas_call(
        kern,
        grid=(B,),
        in_specs=in_specs,
        out_specs=[
            pl.BlockSpec((1, H, W, 81), lambda b: (b, 0, 0, 0)),
            pl.BlockSpec((1, H, W, 4), lambda b: (b, 0, 0, 0)),
        ],
        out_shape=[
            jax.ShapeDtypeStruct((B, H, W, 81), jnp.float32),
            jax.ShapeDtypeStruct((B, H, W, 4), jnp.float32),
        ],
        scratch_shapes=[
            pltpu.VMEM((H + 2, W + 2, _C), jnp.bfloat16),
            pltpu.VMEM((H * W, _C), jnp.float32),
        ],
    )(feat, *args)


def _gs_vec(v):
    return jnp.repeat(v.reshape(_GROUPS, _GSIZE).sum(axis=1), _GSIZE)


def _prep_tower(layers):
    ws, lps = [], []
    for l in layers:
        ws.append(jnp.transpose(l['w'], (2, 3, 1, 0)).reshape(9 * _C, _C))
        b, g, beta = l['b'], l['g'], l['beta']
        lps.append(jnp.stack([b, g, beta, _gs_vec(b), _gs_vec(b * b),
                              jnp.zeros_like(b), jnp.zeros_like(b),
                              jnp.zeros_like(b)]))
    return (jnp.stack(ws).astype(jnp.bfloat16),
            jnp.stack(lps).astype(jnp.float32))


def _prep_head(w):
    co = w.shape[0]
    return jnp.transpose(w, (2, 3, 1, 0)).reshape(9 * _C, co).astype(
        jnp.bfloat16)


def kernel(features, params):
    cls_tw, cls_lp = _prep_tower(params['cls_tower'])
    box_tw, box_lp = _prep_tower(params['bbox_tower'])
    cls_hw = _prep_head(jnp.concatenate(
        [params['cls_logits']['w'], params['centerness']['w']], axis=0))
    cls_hb = jnp.concatenate(
        [params['cls_logits']['b'], params['centerness']['b']])[None, :]
    box_hw = _prep_head(params['bbox_pred']['w'])
    box_hb = params['bbox_pred']['b'][None, :]
    cls_p = (cls_tw, cls_lp, cls_hw, cls_hb)
    box_p = (box_tw, box_lp, box_hw, box_hb)

    logits, bbox, ctr = [], [], []
    for l, f in enumerate(features):
        fx = jnp.transpose(f, (0, 2, 3, 1)).astype(jnp.bfloat16)
        sc = params['scales'][l].reshape(1, 1)
        yc, yb = _run_level(fx, cls_p, box_p, sc)
        logits.append(jnp.transpose(yc[..., :80], (0, 3, 1, 2)))
        ctr.append(jnp.transpose(yc[..., 80:81], (0, 3, 1, 2)))
        bbox.append(jnp.transpose(yb, (0, 3, 1, 2)))
    return tuple(logits), tuple(bbox), tuple(ctr)


# speedup vs baseline: 1.5325x; 1.0847x over previous
import functools
import jax
import jax.numpy as jnp
from jax.experimental import pallas as pl
from jax.experimental.pallas import tpu as pltpu

_C = 256
_GROUPS = 32
_GSIZE = _C // _GROUPS
_EPS = 1e-5


def _group_mat():
    r = jax.lax.broadcasted_iota(jnp.int32, (_C, _C), 0) // _GSIZE
    c = jax.lax.broadcasted_iota(jnp.int32, (_C, _C), 1) // _GSIZE
    return (r == c).astype(jnp.float32)


def _chunks(H, W):
    ch = max(1, min(H, 256 // W))
    return [(h0, min(ch, H - h0)) for h0 in range(0, H, ch)]


def _im2col(pad_ref, h0, ch, W):
    return jnp.concatenate(
        [pad_ref[h0 + k // 3:h0 + k // 3 + ch,
                 k % 3:k % 3 + W, :].reshape(ch * W, _C)
         for k in range(9)], axis=1)


def _tower(feat_ref, tw_ref, lp_ref, hw_ref, hb_ref, out_ref, pad_ref,
           acc_ref, bb_scale, H, W, head_co, bbox):
    N = H * W
    P = _group_mat()
    chunks = _chunks(H, W)

    pad_ref[...] = jnp.zeros_like(pad_ref)
    pad_ref[1:H + 1, 1:W + 1, :] = feat_ref[0]

    for layer in range(tw_ref.shape[0]):
        s = q = None
        for h0, ch in chunks:
            a = jnp.dot(_im2col(pad_ref, h0, ch, W), tw_ref[layer],
                        preferred_element_type=jnp.float32)
            acc_ref[h0 * W:(h0 + ch) * W, :] = a
            cs = jnp.sum(a, axis=0, keepdims=True)
            cq = jnp.sum(a * a, axis=0, keepdims=True)
            s = cs if s is None else s + cs
            q = cq if q is None else q + cq
        lp = lp_ref[layer]
        b, gamma, beta = lp[0:1], lp[1:2], lp[2:3]
        gsb, gsb2 = lp[3:4], lp[4:5]
        stats = jnp.concatenate([s, q, b * s], axis=0)
        gs = jnp.dot(stats, P, preferred_element_type=jnp.float32)
        inv_n = 1.0 / (_GSIZE * N)
        mu = (gs[0:1] + N * gsb) * inv_n
        ey2 = (gs[1:2] + 2.0 * gs[2:3] + N * gsb2) * inv_n
        rstd = jax.lax.rsqrt(ey2 - mu * mu + _EPS)
        sc = rstd * gamma
        sh = (b - mu) * sc + beta
        for h0, ch in chunks:
            a = acc_ref[h0 * W:(h0 + ch) * W, :]
            x = jnp.maximum(a * sc + sh, 0.0).astype(jnp.bfloat16)
            pad_ref[h0 + 1:h0 + ch + 1, 1:W + 1, :] = x.reshape(ch, W, _C)

    for h0, ch in chunks:
        y = jnp.dot(_im2col(pad_ref, h0, ch, W), hw_ref[...],
                    preferred_element_type=jnp.float32) + hb_ref[0:1]
        if bbox:
            y = jnp.exp(y * bb_scale)
        out_ref[0, h0:h0 + ch] = y.reshape(ch, W, head_co)


def _mega_kernel(*refs, dims):
    nl = len(dims)
    feats = refs[0:nl]
    ctw, clp, chw, chb, btw, blp, bhw, bhb, scs = refs[nl:nl + 9]
    outs = refs[nl + 9:nl + 9 + 2 * nl]
    scr = refs[nl + 9 + 2 * nl:]
    pads, accs = scr[0:nl], scr[nl:2 * nl]
    for l, (H, W) in enumerate(dims):
        _tower(feats[l], ctw, clp, chw, chb, outs[2 * l],
               pads[l], accs[l], None, H, W, 81, False)
        _tower(feats[l], btw, blp, bhw, bhb, outs[2 * l + 1],
               pads[l], accs[l], scs[l:l + 1, 0:1], H, W, 4, True)


def _run_all(feats, cls_p, box_p, scales):
    B = feats[0].shape[0]
    dims = [(f.shape[1], f.shape[2]) for f in feats]
    kern = functools.partial(_mega_kernel, dims=dims)
    full = lambda a: pl.BlockSpec(a.shape, lambda b: (0,) * a.ndim)
    wargs = list(cls_p) + list(box_p) + [scales]
    in_specs = ([pl.BlockSpec((1, H, W, _C), lambda b: (b, 0, 0, 0))
                 for (H, W) in dims] + [full(a) for a in wargs])
    out_specs, out_shape, scratch = [], [], []
    for (H, W) in dims:
        for co in (81, 4):
            out_specs.append(
                pl.BlockSpec((1, H, W, co), lambda b: (b, 0, 0, 0)))
            out_shape.append(
                jax.ShapeDtypeStruct((B, H, W, co), jnp.float32))
    for (H, W) in dims:
        scratch.append(pltpu.VMEM((H + 2, W + 2, _C), jnp.bfloat16))
    for (H, W) in dims:
        scratch.append(pltpu.VMEM((H * W, _C), jnp.float32))
    return pl.pallas_call(
        kern,
        grid=(B,),
        in_specs=in_specs,
        out_specs=out_specs,
        out_shape=out_shape,
        scratch_shapes=scratch,
    )(*feats, *wargs)


def _gs_vec(v):
    return jnp.repeat(v.reshape(_GROUPS, _GSIZE).sum(axis=1), _GSIZE)


def _prep_tower(layers):
    ws, lps = [], []
    for l in layers:
        ws.append(jnp.transpose(l['w'], (2, 3, 1, 0)).reshape(9 * _C, _C))
        b, g, beta = l['b'], l['g'], l['beta']
        lps.append(jnp.stack([b, g, beta, _gs_vec(b), _gs_vec(b * b),
                              jnp.zeros_like(b), jnp.zeros_like(b),
                              jnp.zeros_like(b)]))
    return (jnp.stack(ws).astype(jnp.bfloat16),
            jnp.stack(lps).astype(jnp.float32))


def _prep_head(w):
    co = w.shape[0]
    return jnp.transpose(w, (2, 3, 1, 0)).reshape(9 * _C, co).astype(
        jnp.bfloat16)


def kernel(features, params):
    cls_tw, cls_lp = _prep_tower(params['cls_tower'])
    box_tw, box_lp = _prep_tower(params['bbox_tower'])
    cls_hw = _prep_head(jnp.concatenate(
        [params['cls_logits']['w'], params['centerness']['w']], axis=0))
    cls_hb = jnp.concatenate(
        [params['cls_logits']['b'], params['centerness']['b']])[None, :]
    box_hw = _prep_head(params['bbox_pred']['w'])
    box_hb = params['bbox_pred']['b'][None, :]
    cls_p = (cls_tw, cls_lp, cls_hw, cls_hb)
    box_p = (box_tw, box_lp, box_hw, box_hb)
    scales = jnp.stack(
        [params['scales'][l].reshape(1) for l in range(len(features))])

    feats = [jnp.transpose(f, (0, 2, 3, 1)).astype(jnp.bfloat16)
             for f in features]
    ys = _run_all(feats, cls_p, box_p, scales.astype(jnp.float32))

    logits, bbox, ctr = [], [], []
    for l in range(len(features)):
        yc, yb = ys[2 * l], ys[2 * l + 1]
        logits.append(jnp.transpose(yc[..., :80], (0, 3, 1, 2)))
        ctr.append(jnp.transpose(yc[..., 80:81], (0, 3, 1, 2)))
        bbox.append(jnp.transpose(yb, (0, 3, 1, 2)))
    return tuple(logits), tuple(bbox), tuple(ctr)


# speedup vs baseline: 1.5347x; 1.0014x over previous
import functools
import jax
import jax.numpy as jnp
from jax.experimental import pallas as pl
from jax.experimental.pallas import tpu as pltpu

_C = 256
_GROUPS = 32
_GSIZE = _C // _GROUPS
_EPS = 1e-5


def _group_mat():
    r = jax.lax.broadcasted_iota(jnp.int32, (_C, _C), 0) // _GSIZE
    c = jax.lax.broadcasted_iota(jnp.int32, (_C, _C), 1) // _GSIZE
    return (r == c).astype(jnp.float32)


def _chunks(H, W):
    ch = max(1, min(H, 256 // W))
    return [(h0, min(ch, H - h0)) for h0 in range(0, H, ch)]


def _im2col(pad_ref, h0, ch, W):
    return jnp.concatenate(
        [pad_ref[h0 + k // 3:h0 + k // 3 + ch,
                 k % 3:k % 3 + W, :].reshape(ch * W, _C)
         for k in range(9)], axis=1)


def _tower(feat_ref, tw_ref, lp_ref, hw_ref, hb_ref, out_ref, pad_ref,
           acc_ref, bb_scale, H, W, head_co, bbox):
    N = H * W
    P = _group_mat()
    chunks = _chunks(H, W)

    pad_ref[...] = jnp.zeros_like(pad_ref)
    pad_ref[1:H + 1, 1:W + 1, :] = feat_ref[0]

    for layer in range(tw_ref.shape[0]):
        s = q = None
        for h0, ch in chunks:
            a = jnp.dot(_im2col(pad_ref, h0, ch, W), tw_ref[layer],
                        preferred_element_type=jnp.float32)
            acc_ref[h0 * W:(h0 + ch) * W, :] = a
            cs = jnp.sum(a, axis=0, keepdims=True)
            cq = jnp.sum(a * a, axis=0, keepdims=True)
            s = cs if s is None else s + cs
            q = cq if q is None else q + cq
        lp = lp_ref[layer]
        b, gamma, beta = lp[0:1], lp[1:2], lp[2:3]
        gsb, gsb2 = lp[3:4], lp[4:5]
        stats = jnp.concatenate([s, q, b * s], axis=0)
        gs = jnp.dot(stats, P, preferred_element_type=jnp.float32)
        inv_n = 1.0 / (_GSIZE * N)
        mu = (gs[0:1] + N * gsb) * inv_n
        ey2 = (gs[1:2] + 2.0 * gs[2:3] + N * gsb2) * inv_n
        rstd = jax.lax.rsqrt(ey2 - mu * mu + _EPS)
        sc = rstd * gamma
        sh = (b - mu) * sc + beta
        for h0, ch in chunks:
            a = acc_ref[h0 * W:(h0 + ch) * W, :]
            x = jnp.maximum(a * sc + sh, 0.0).astype(jnp.bfloat16)
            pad_ref[h0 + 1:h0 + ch + 1, 1:W + 1, :] = x.reshape(ch, W, _C)

    for h0, ch in chunks:
        y = jnp.dot(_im2col(pad_ref, h0, ch, W), hw_ref[...],
                    preferred_element_type=jnp.float32) + hb_ref[0:1]
        if bbox:
            y = jnp.exp(y * bb_scale)
        out_ref[0, h0:h0 + ch] = y.reshape(ch, W, head_co)


def _mega_kernel(*refs, dims):
    nl = len(dims)
    feats = refs[0:nl]
    ctw, clp, chw, chb, btw, blp, bhw, bhb, scs = refs[nl:nl + 9]
    outs = refs[nl + 9:nl + 9 + 2 * nl]
    scr = refs[nl + 9 + 2 * nl:]
    pads, accs = scr[0:nl], scr[nl:2 * nl]
    for l, (H, W) in enumerate(dims):
        _tower(feats[l], ctw, clp, chw, chb, outs[2 * l],
               pads[l], accs[l], None, H, W, 81, False)
        _tower(feats[l], btw, blp, bhw, bhb, outs[2 * l + 1],
               pads[l], accs[l], scs[l:l + 1, 0:1], H, W, 4, True)


def _run_all(feats, cls_p, box_p, scales):
    B = feats[0].shape[0]
    dims = [(f.shape[1], f.shape[2]) for f in feats]
    kern = functools.partial(_mega_kernel, dims=dims)
    full = lambda a: pl.BlockSpec(a.shape, lambda b: (0,) * a.ndim)
    wargs = list(cls_p) + list(box_p) + [scales]
    in_specs = ([pl.BlockSpec((1, H, W, _C), lambda b: (b, 0, 0, 0))
                 for (H, W) in dims] + [full(a) for a in wargs])
    out_specs, out_shape, scratch = [], [], []
    for (H, W) in dims:
        for co in (81, 4):
            out_specs.append(
                pl.BlockSpec((1, H, W, co), lambda b: (b, 0, 0, 0)))
            out_shape.append(
                jax.ShapeDtypeStruct((B, H, W, co), jnp.float32))
    for (H, W) in dims:
        scratch.append(pltpu.VMEM((H + 2, W + 2, _C), jnp.bfloat16))
    for (H, W) in dims:
        scratch.append(pltpu.VMEM((H * W, _C), jnp.float32))
    return pl.pallas_call(
        kern,
        grid=(B,),
        in_specs=in_specs,
        out_specs=out_specs,
        out_shape=out_shape,
        scratch_shapes=scratch,
        compiler_params=pltpu.CompilerParams(
            dimension_semantics=("parallel",)),
    )(*feats, *wargs)


def _gs_vec(v):
    return jnp.repeat(v.reshape(_GROUPS, _GSIZE).sum(axis=1), _GSIZE)


def _prep_tower(layers):
    ws, lps = [], []
    for l in layers:
        ws.append(jnp.transpose(l['w'], (2, 3, 1, 0)).reshape(9 * _C, _C))
        b, g, beta = l['b'], l['g'], l['beta']
        lps.append(jnp.stack([b, g, beta, _gs_vec(b), _gs_vec(b * b),
                              jnp.zeros_like(b), jnp.zeros_like(b),
                              jnp.zeros_like(b)]))
    return (jnp.stack(ws).astype(jnp.bfloat16),
            jnp.stack(lps).astype(jnp.float32))


def _prep_head(w):
    co = w.shape[0]
    return jnp.transpose(w, (2, 3, 1, 0)).reshape(9 * _C, co).astype(
        jnp.bfloat16)


def kernel(features, params):
    cls_tw, cls_lp = _prep_tower(params['cls_tower'])
    box_tw, box_lp = _prep_tower(params['bbox_tower'])
    cls_hw = _prep_head(jnp.concatenate(
        [params['cls_logits']['w'], params['centerness']['w']], axis=0))
    cls_hb = jnp.concatenate(
        [params['cls_logits']['b'], params['centerness']['b']])[None, :]
    box_hw = _prep_head(params['bbox_pred']['w'])
    box_hb = params['bbox_pred']['b'][None, :]
    cls_p = (cls_tw, cls_lp, cls_hw, cls_hb)
    box_p = (box_tw, box_lp, box_hw, box_hb)
    scales = jnp.stack(
        [params['scales'][l].reshape(1) for l in range(len(features))])

    feats = [jnp.transpose(f, (0, 2, 3, 1)).astype(jnp.bfloat16)
             for f in features]
    ys = _run_all(feats, cls_p, box_p, scales.astype(jnp.float32))

    logits, bbox, ctr = [], [], []
    for l in range(len(features)):
        yc, yb = ys[2 * l], ys[2 * l + 1]
        logits.append(jnp.transpose(yc[..., :80], (0, 3, 1, 2)))
        ctr.append(jnp.transpose(yc[..., 80:81], (0, 3, 1, 2)))
        bbox.append(jnp.transpose(yb, (0, 3, 1, 2)))
    return tuple(logits), tuple(bbox), tuple(ctr)


# speedup vs baseline: 1.5539x; 1.0126x over previous
import functools
import jax
import jax.numpy as jnp
from jax.experimental import pallas as pl
from jax.experimental.pallas import tpu as pltpu

_C = 256
_GROUPS = 32
_GSIZE = _C // _GROUPS
_EPS = 1e-5


def _group_mat():
    r = jax.lax.broadcasted_iota(jnp.int32, (_C, _C), 0) // _GSIZE
    c = jax.lax.broadcasted_iota(jnp.int32, (_C, _C), 1) // _GSIZE
    return (r == c).astype(jnp.float32)


def _chunks(H, W):
    ch = max(1, min(H, 256 // W))
    return [(h0, min(ch, H - h0)) for h0 in range(0, H, ch)]


def _im2col(b_ref, h0, ch, W):
    return jnp.concatenate(
        [b_ref[k % 3, (h0 + k // 3) * W:(h0 + k // 3 + ch) * W, :]
         for k in range(9)], axis=1)


def _repack(b_ref, pad_ref, H, W):
    for kw in range(3):
        b_ref[kw] = pad_ref[0:H + 2, kw:kw + W, :].reshape((H + 2) * W, _C)


def _tower(feat_ref, tw_ref, lp_ref, hw_ref, hb_ref, out_ref, pad_ref,
           b_ref, acc_ref, bb_scale, H, W, head_co, bbox):
    N = H * W
    P = _group_mat()
    chunks = _chunks(H, W)

    pad_ref[...] = jnp.zeros_like(pad_ref)
    pad_ref[1:H + 1, 1:W + 1, :] = feat_ref[0]

    for layer in range(tw_ref.shape[0]):
        _repack(b_ref, pad_ref, H, W)
        s = q = None
        for h0, ch in chunks:
            a = jnp.dot(_im2col(b_ref, h0, ch, W), tw_ref[layer],
                        preferred_element_type=jnp.float32)
            acc_ref[h0 * W:(h0 + ch) * W, :] = a
            cs = jnp.sum(a, axis=0, keepdims=True)
            cq = jnp.sum(a * a, axis=0, keepdims=True)
            s = cs if s is None else s + cs
            q = cq if q is None else q + cq
        lp = lp_ref[layer]
        b, gamma, beta = lp[0:1], lp[1:2], lp[2:3]
        gsb, gsb2 = lp[3:4], lp[4:5]
        stats = jnp.concatenate([s, q, b * s], axis=0)
        gs = jnp.dot(stats, P, preferred_element_type=jnp.float32)
        inv_n = 1.0 / (_GSIZE * N)
        mu = (gs[0:1] + N * gsb) * inv_n
        ey2 = (gs[1:2] + 2.0 * gs[2:3] + N * gsb2) * inv_n
        rstd = jax.lax.rsqrt(ey2 - mu * mu + _EPS)
        sc = rstd * gamma
        sh = (b - mu) * sc + beta
        for h0, ch in chunks:
            a = acc_ref[h0 * W:(h0 + ch) * W, :]
            x = jnp.maximum(a * sc + sh, 0.0).astype(jnp.bfloat16)
            pad_ref[h0 + 1:h0 + ch + 1, 1:W + 1, :] = x.reshape(ch, W, _C)

    _repack(b_ref, pad_ref, H, W)
    for h0, ch in chunks:
        y = jnp.dot(_im2col(b_ref, h0, ch, W), hw_ref[...],
                    preferred_element_type=jnp.float32) + hb_ref[0:1]
        if bbox:
            y = jnp.exp(y * bb_scale)
        out_ref[0, h0:h0 + ch] = y.reshape(ch, W, head_co)


def _mega_kernel(*refs, dims):
    nl = len(dims)
    feats = refs[0:nl]
    ctw, clp, chw, chb, btw, blp, bhw, bhb, scs = refs[nl:nl + 9]
    outs = refs[nl + 9:nl + 9 + 2 * nl]
    scr = refs[nl + 9 + 2 * nl:]
    pads, bufs, accs = scr[0:nl], scr[nl:2 * nl], scr[2 * nl:3 * nl]
    for l, (H, W) in enumerate(dims):
        _tower(feats[l], ctw, clp, chw, chb, outs[2 * l],
               pads[l], bufs[l], accs[l], None, H, W, 81, False)
        _tower(feats[l], btw, blp, bhw, bhb, outs[2 * l + 1],
               pads[l], bufs[l], accs[l], scs[l:l + 1, 0:1], H, W, 4,
               True)


def _run_all(feats, cls_p, box_p, scales):
    B = feats[0].shape[0]
    dims = [(f.shape[1], f.shape[2]) for f in feats]
    kern = functools.partial(_mega_kernel, dims=dims)
    full = lambda a: pl.BlockSpec(a.shape, lambda b: (0,) * a.ndim)
    wargs = list(cls_p) + list(box_p) + [scales]
    in_specs = ([pl.BlockSpec((1, H, W, _C), lambda b: (b, 0, 0, 0))
                 for (H, W) in dims] + [full(a) for a in wargs])
    out_specs, out_shape, scratch = [], [], []
    for (H, W) in dims:
        for co in (81, 4):
            out_specs.append(
                pl.BlockSpec((1, H, W, co), lambda b: (b, 0, 0, 0)))
            out_shape.append(
                jax.ShapeDtypeStruct((B, H, W, co), jnp.float32))
    for (H, W) in dims:
        scratch.append(pltpu.VMEM((H + 2, W + 2, _C), jnp.bfloat16))
    for (H, W) in dims:
        scratch.append(pltpu.VMEM((3, (H + 2) * W, _C), jnp.bfloat16))
    for (H, W) in dims:
        scratch.append(pltpu.VMEM((H * W, _C), jnp.float32))
    return pl.pallas_call(
        kern,
        grid=(B,),
        in_specs=in_specs,
        out_specs=out_specs,
        out_shape=out_shape,
        scratch_shapes=scratch,
        compiler_params=pltpu.CompilerParams(
            dimension_semantics=("parallel",)),
    )(*feats, *wargs)


def _gs_vec(v):
    return jnp.repeat(v.reshape(_GROUPS, _GSIZE).sum(axis=1), _GSIZE)


def _prep_tower(layers):
    ws, lps = [], []
    for l in layers:
        ws.append(jnp.transpose(l['w'], (2, 3, 1, 0)).reshape(9 * _C, _C))
        b, g, beta = l['b'], l['g'], l['beta']
        lps.append(jnp.stack([b, g, beta, _gs_vec(b), _gs_vec(b * b),
                              jnp.zeros_like(b), jnp.zeros_like(b),
                              jnp.zeros_like(b)]))
    return (jnp.stack(ws).astype(jnp.bfloat16),
            jnp.stack(lps).astype(jnp.float32))


def _prep_head(w):
    co = w.shape[0]
    return jnp.transpose(w, (2, 3, 1, 0)).reshape(9 * _C, co).astype(
        jnp.bfloat16)


def kernel(features, params):
    cls_tw, cls_lp = _prep_tower(params['cls_tower'])
    box_tw, box_lp = _prep_tower(params['bbox_tower'])
    cls_hw = _prep_head(jnp.concatenate(
        [params['cls_logits']['w'], params['centerness']['w']], axis=0))
    cls_hb = jnp.concatenate(
        [params['cls_logits']['b'], params['centerness']['b']])[None, :]
    box_hw = _prep_head(params['bbox_pred']['w'])
    box_hb = params['bbox_pred']['b'][None, :]
    cls_p = (cls_tw, cls_lp, cls_hw, cls_hb)
    box_p = (box_tw, box_lp, box_hw, box_hb)
    scales = jnp.stack(
        [params['scales'][l].reshape(1) for l in range(len(features))])

    feats = [jnp.transpose(f, (0, 2, 3, 1)).astype(jnp.bfloat16)
             for f in features]
    ys = _run_all(feats, cls_p, box_p, scales.astype(jnp.float32))

    logits, bbox, ctr = [], [], []
    for l in range(len(features)):
        yc, yb = ys[2 * l], ys[2 * l + 1]
        logits.append(jnp.transpose(yc[..., :80], (0, 3, 1, 2)))
        ctr.append(jnp.transpose(yc[..., 80:81], (0, 3, 1, 2)))
        bbox.append(jnp.transpose(yb, (0, 3, 1, 2)))
    return tuple(logits), tuple(bbox), tuple(ctr)


# speedup vs baseline: 1.5541x; 1.0001x over previous
import functools
import jax
import jax.numpy as jnp
from jax.experimental import pallas as pl
from jax.experimental.pallas import tpu as pltpu

_C = 256
_GROUPS = 32
_GSIZE = _C // _GROUPS
_EPS = 1e-5


def _group_mat():
    r = jax.lax.broadcasted_iota(jnp.int32, (_C, _C), 0) // _GSIZE
    c = jax.lax.broadcasted_iota(jnp.int32, (_C, _C), 1) // _GSIZE
    return (r == c).astype(jnp.float32)


def _chunks(H, W):
    ch = max(1, min(H, 256 // W))
    return [(h0, min(ch, H - h0)) for h0 in range(0, H, ch)]


def _conv_chunk(b_ref, w_ref, h0, ch, W, layer=None):
    a = None
    for k in range(9):
        win = b_ref[k % 3, (h0 + k // 3) * W:(h0 + k // 3 + ch) * W, :]
        wk = (w_ref[layer, k * _C:(k + 1) * _C, :] if layer is not None
              else w_ref[k * _C:(k + 1) * _C, :])
        t = jnp.dot(win, wk, preferred_element_type=jnp.float32)
        a = t if a is None else a + t
    return a


def _repack(b_ref, pad_ref, H, W):
    for kw in range(3):
        b_ref[kw] = pad_ref[0:H + 2, kw:kw + W, :].reshape((H + 2) * W, _C)


def _tower(feat_ref, tw_ref, lp_ref, hw_ref, hb_ref, out_ref, pad_ref,
           b_ref, acc_ref, bb_scale, H, W, head_co, bbox):
    N = H * W
    P = _group_mat()
    chunks = _chunks(H, W)

    pad_ref[...] = jnp.zeros_like(pad_ref)
    pad_ref[1:H + 1, 1:W + 1, :] = feat_ref[0]

    for layer in range(tw_ref.shape[0]):
        _repack(b_ref, pad_ref, H, W)
        s = q = None
        for h0, ch in chunks:
            a = _conv_chunk(b_ref, tw_ref, h0, ch, W, layer=layer)
            acc_ref[h0 * W:(h0 + ch) * W, :] = a
            cs = jnp.sum(a, axis=0, keepdims=True)
            cq = jnp.sum(a * a, axis=0, keepdims=True)
            s = cs if s is None else s + cs
            q = cq if q is None else q + cq
        lp = lp_ref[layer]
        b, gamma, beta = lp[0:1], lp[1:2], lp[2:3]
        gsb, gsb2 = lp[3:4], lp[4:5]
        stats = jnp.concatenate([s, q, b * s], axis=0)
        gs = jnp.dot(stats, P, preferred_element_type=jnp.float32)
        inv_n = 1.0 / (_GSIZE * N)
        mu = (gs[0:1] + N * gsb) * inv_n
        ey2 = (gs[1:2] + 2.0 * gs[2:3] + N * gsb2) * inv_n
        rstd = jax.lax.rsqrt(ey2 - mu * mu + _EPS)
        sc = rstd * gamma
        sh = (b - mu) * sc + beta
        for h0, ch in chunks:
            a = acc_ref[h0 * W:(h0 + ch) * W, :]
            x = jnp.maximum(a * sc + sh, 0.0).astype(jnp.bfloat16)
            pad_ref[h0 + 1:h0 + ch + 1, 1:W + 1, :] = x.reshape(ch, W, _C)

    _repack(b_ref, pad_ref, H, W)
    for h0, ch in chunks:
        y = _conv_chunk(b_ref, hw_ref, h0, ch, W) + hb_ref[0:1]
        if bbox:
            y = jnp.exp(y * bb_scale)
        out_ref[0, h0:h0 + ch] = y.reshape(ch, W, head_co)


def _mega_kernel(*refs, dims):
    nl = len(dims)
    feats = refs[0:nl]
    ctw, clp, chw, chb, btw, blp, bhw, bhb, scs = refs[nl:nl + 9]
    outs = refs[nl + 9:nl + 9 + 2 * nl]
    scr = refs[nl + 9 + 2 * nl:]
    pads, bufs, accs = scr[0:nl], scr[nl:2 * nl], scr[2 * nl:3 * nl]
    for l, (H, W) in enumerate(dims):
        _tower(feats[l], ctw, clp, chw, chb, outs[2 * l],
               pads[l], bufs[l], accs[l], None, H, W, 81, False)
        _tower(feats[l], btw, blp, bhw, bhb, outs[2 * l + 1],
               pads[l], bufs[l], accs[l], scs[l:l + 1, 0:1], H, W, 4,
               True)


def _run_all(feats, cls_p, box_p, scales):
    B = feats[0].shape[0]
    dims = [(f.shape[1], f.shape[2]) for f in feats]
    kern = functools.partial(_mega_kernel, dims=dims)
    full = lambda a: pl.BlockSpec(a.shape, lambda b: (0,) * a.ndim)
    wargs = list(cls_p) + list(box_p) + [scales]
    in_specs = ([pl.BlockSpec((1, H, W, _C), lambda b: (b, 0, 0, 0))
                 for (H, W) in dims] + [full(a) for a in wargs])
    out_specs, out_shape, scratch = [], [], []
    for (H, W) in dims:
        for co in (81, 4):
            out_specs.append(
                pl.BlockSpec((1, H, W, co), lambda b: (b, 0, 0, 0)))
            out_shape.append(
                jax.ShapeDtypeStruct((B, H, W, co), jnp.float32))
    for (H, W) in dims:
        scratch.append(pltpu.VMEM((H + 2, W + 2, _C), jnp.bfloat16))
    for (H, W) in dims:
        scratch.append(pltpu.VMEM((3, (H + 2) * W, _C), jnp.bfloat16))
    for (H, W) in dims:
        scratch.append(pltpu.VMEM((H * W, _C), jnp.float32))
    return pl.pallas_call(
        kern,
        grid=(B,),
        in_specs=in_specs,
        out_specs=out_specs,
        out_shape=out_shape,
        scratch_shapes=scratch,
        compiler_params=pltpu.CompilerParams(
            dimension_semantics=("parallel",)),
    )(*feats, *wargs)


def _gs_vec(v):
    return jnp.repeat(v.reshape(_GROUPS, _GSIZE).sum(axis=1), _GSIZE)


def _prep_tower(layers):
    ws, lps = [], []
    for l in layers:
        ws.append(jnp.transpose(l['w'], (2, 3, 1, 0)).reshape(9 * _C, _C))
        b, g, beta = l['b'], l['g'], l['beta']
        lps.append(jnp.stack([b, g, beta, _gs_vec(b), _gs_vec(b * b),
                              jnp.zeros_like(b), jnp.zeros_like(b),
                              jnp.zeros_like(b)]))
    return (jnp.stack(ws).astype(jnp.bfloat16),
            jnp.stack(lps).astype(jnp.float32))


def _prep_head(w):
    co = w.shape[0]
    return jnp.transpose(w, (2, 3, 1, 0)).reshape(9 * _C, co).astype(
        jnp.bfloat16)


def kernel(features, params):
    cls_tw, cls_lp = _prep_tower(params['cls_tower'])
    box_tw, box_lp = _prep_tower(params['bbox_tower'])
    cls_hw = _prep_head(jnp.concatenate(
        [params['cls_logits']['w'], params['centerness']['w']], axis=0))
    cls_hb = jnp.concatenate(
        [params['cls_logits']['b'], params['centerness']['b']])[None, :]
    box_hw = _prep_head(params['bbox_pred']['w'])
    box_hb = params['bbox_pred']['b'][None, :]
    cls_p = (cls_tw, cls_lp, cls_hw, cls_hb)
    box_p = (box_tw, box_lp, box_hw, box_hb)
    scales = jnp.stack(
        [params['scales'][l].reshape(1) for l in range(len(features))])

    feats = [jnp.transpose(f, (0, 2, 3, 1)).astype(jnp.bfloat16)
             for f in features]
    ys = _run_all(feats, cls_p, box_p, scales.astype(jnp.float32))

    logits, bbox, ctr = [], [], []
    for l in range(len(features)):
        yc, yb = ys[2 * l], ys[2 * l + 1]
        logits.append(jnp.transpose(yc[..., :80], (0, 3, 1, 2)))
        ctr.append(jnp.transpose(yc[..., 80:81], (0, 3, 1, 2)))
        bbox.append(jnp.transpose(yb, (0, 3, 1, 2)))
    return tuple(logits), tuple(bbox), tuple(ctr)
